# baseline (speedup 1.0000x reference)
import functools

import jax
import jax.numpy as jnp
from jax.experimental import pallas as pl

NH = 8


def _mm_kernel(x_ref, w_ref, b_ref, o_ref):
    o_ref[...] = (
        jnp.dot(x_ref[...], w_ref[...], preferred_element_type=jnp.float32)
        + b_ref[...]
    )


@functools.partial(jax.jit, static_argnames=("bm",))
def _matmul_bias(x, w, b, bm=512):
    m, k = x.shape
    n = w.shape[1]
    mp = ((m + bm - 1) // bm) * bm
    xp = jnp.pad(x, ((0, mp - m), (0, 0)))
    out = pl.pallas_call(
        _mm_kernel,
        grid=(mp // bm,),
        in_specs=[
            pl.BlockSpec((bm, k), lambda i: (i, 0)),
            pl.BlockSpec((k, n), lambda i: (0, 0)),
            pl.BlockSpec((1, n), lambda i: (0, 0)),
        ],
        out_specs=pl.BlockSpec((bm, n), lambda i: (i, 0)),
        out_shape=jax.ShapeDtypeStruct((mp, n), jnp.float32),
    )(xp, w, b.reshape(1, n))
    return out[:m]


def kernel(h, edge_index, weights):
    n_nodes, hdim = h.shape
    hd = hdim // NH
    scale = hd ** -0.5
    nlayers = sum(1 for kk in weights if kk.endswith("_Wq"))
    row = edge_index[0]
    col = edge_index[1]

    x = h
    for l in range(nlayers):
        wqkv = jnp.concatenate(
            [weights[f"l{l}_Wq"], weights[f"l{l}_Wk"], weights[f"l{l}_Wv"]],
            axis=1,
        )
        bqkv = jnp.concatenate(
            [weights[f"l{l}_bq"], weights[f"l{l}_bk"], weights[f"l{l}_bv"]]
        )
        qkv = _matmul_bias(x, wqkv, bqkv)
        q = qkv[:, :hdim].reshape(n_nodes, hd, NH) * scale
        k = qkv[:, hdim : 2 * hdim].reshape(n_nodes, hd, NH)
        v = qkv[:, 2 * hdim :].reshape(n_nodes, hd, NH)
        scores = jnp.einsum("edh,edh->eh", q[row], k[col])
        smax = jax.ops.segment_max(scores, row, num_segments=n_nodes)
        smax = jnp.where(jnp.isfinite(smax), smax, 0.0)
        ex = jnp.exp(scores - smax[row])
        denom = jax.ops.segment_sum(ex, row, num_segments=n_nodes)
        attn = ex / (denom[row] + 1e-9)
        out = jax.ops.segment_sum(
            attn[:, None, :] * v[col], row, num_segments=n_nodes
        )
        x = _matmul_bias(
            out.reshape(n_nodes, hdim), weights[f"l{l}_Wo"], weights[f"l{l}_bo"]
        )
    wp = jnp.pad(weights["Wp"], ((0, 0), (0, 127)))
    bp = jnp.pad(weights["bp"], (0, 127))
    return _matmul_bias(x, wp, bp)[:, :1]


# speedup vs baseline: 18.3280x; 18.3280x over previous
import functools

import jax
import jax.numpy as jnp
from jax import lax
from jax.experimental import pallas as pl
from jax.experimental.pallas import tpu as pltpu
from jax.experimental.pallas import tpu_sc as plsc

NH = 8
LANES = 16
CHUNK = 48
NT = 16
NSC = 2
CLAMP = 60.0


def _mm_kernel(x_ref, w_ref, b_ref, o_ref):
    o_ref[...] = (
        jnp.dot(x_ref[...], w_ref[...], preferred_element_type=jnp.float32)
        + b_ref[...]
    )


def _matmul_bias(x, w, b, bm=512):
    m, k = x.shape
    n = w.shape[1]
    mp = ((m + bm - 1) // bm) * bm
    xp = jnp.pad(x, ((0, mp - m), (0, 0)))
    out = pl.pallas_call(
        _mm_kernel,
        grid=(mp // bm,),
        in_specs=[
            pl.BlockSpec((bm, k), lambda i: (i, 0)),
            pl.BlockSpec((k, n), lambda i: (0, 0)),
            pl.BlockSpec((1, n), lambda i: (0, 0)),
        ],
        out_specs=pl.BlockSpec((bm, n), lambda i: (i, 0)),
        out_shape=jax.ShapeDtypeStruct((mp, n), jnp.float32),
    )(xp, w, b.reshape(1, n))
    return out[:m]


def _proj3_kernel(hdim, x_ref, w_ref, b_ref, q_ref, k_ref, v_ref):
    y = (
        jnp.dot(x_ref[...], w_ref[...], preferred_element_type=jnp.float32)
        + b_ref[...]
    )
    q_ref[...] = y[:, :hdim]
    k_ref[...] = y[:, hdim : 2 * hdim]
    v_ref[...] = y[:, 2 * hdim :]


def _norm3_kernel(hdim, xv_ref, xd_ref, w_ref, b_ref, q_ref, k_ref, v_ref):
    inv = 1.0 / (xd_ref[...] + 1e-9)
    xn = xv_ref[...] * jnp.tile(inv, (1, hdim // LANES))
    y = jnp.dot(xn, w_ref[...], preferred_element_type=jnp.float32) + b_ref[...]
    q_ref[...] = y[:, :hdim]
    k_ref[...] = y[:, hdim : 2 * hdim]
    v_ref[...] = y[:, 2 * hdim :]


def _normf_kernel(hdim, xv_ref, xd_ref, w_ref, b_ref, o_ref):
    inv = 1.0 / (xd_ref[...] + 1e-9)
    xn = xv_ref[...] * jnp.tile(inv, (1, hdim // LANES))
    o_ref[...] = (
        jnp.dot(xn, w_ref[...], preferred_element_type=jnp.float32) + b_ref[...]
    )


def _proj3(x, w, b, hdim, bm=512):
    m = x.shape[0]
    n = w.shape[1]
    sd = jax.ShapeDtypeStruct((m, hdim), jnp.float32)
    return pl.pallas_call(
        functools.partial(_proj3_kernel, hdim),
        grid=(m // bm,),
        in_specs=[
            pl.BlockSpec((bm, x.shape[1]), lambda i: (i, 0)),
            pl.BlockSpec((x.shape[1], n), lambda i: (0, 0)),
            pl.BlockSpec((1, n), lambda i: (0, 0)),
        ],
        out_specs=[pl.BlockSpec((bm, hdim), lambda i: (i, 0))] * 3,
        out_shape=[sd, sd, sd],
    )(x, w, b.reshape(1, n))


def _norm3(xv, xd, w, b, hdim, bm=512):
    m = xv.shape[0]
    n = w.shape[1]
    sd = jax.ShapeDtypeStruct((m, hdim), jnp.float32)
    return pl.pallas_call(
        functools.partial(_norm3_kernel, hdim),
        grid=(m // bm,),
        in_specs=[
            pl.BlockSpec((bm, hdim), lambda i: (i, 0)),
            pl.BlockSpec((bm, LANES), lambda i: (i, 0)),
            pl.BlockSpec((hdim, n), lambda i: (0, 0)),
            pl.BlockSpec((1, n), lambda i: (0, 0)),
        ],
        out_specs=[pl.BlockSpec((bm, hdim), lambda i: (i, 0))] * 3,
        out_shape=[sd, sd, sd],
    )(xv, xd, w, b.reshape(1, n))


def _normf(xv, xd, w, b, hdim, bm=512):
    m = xv.shape[0]
    n = w.shape[1]
    return pl.pallas_call(
        functools.partial(_normf_kernel, hdim),
        grid=(m // bm,),
        in_specs=[
            pl.BlockSpec((bm, hdim), lambda i: (i, 0)),
            pl.BlockSpec((bm, LANES), lambda i: (i, 0)),
            pl.BlockSpec((hdim, n), lambda i: (0, 0)),
            pl.BlockSpec((1, n), lambda i: (0, 0)),
        ],
        out_specs=pl.BlockSpec((bm, n), lambda i: (i, 0)),
        out_shape=jax.ShapeDtypeStruct((m, n), jnp.float32),
    )(xv, xd, w, b.reshape(1, n))


def _make_sc_attention(n_pad, hdim, e_pad, n_edges):
    n_per_sc = n_pad // NSC
    rows_per_tile = n_per_sc // NT
    nslabs = n_per_sc // CHUNK + 1
    acc_rows = nslabs * CHUNK
    trash = n_per_sc
    cpt = e_pad // (NT * CHUNK)
    ept = cpt * CHUNK

    mesh = plsc.VectorSubcoreMesh(core_axis_name="c", subcore_axis_name="s")

    @functools.partial(
        pl.kernel,
        out_type=[
            jax.ShapeDtypeStruct((n_pad, hdim), jnp.float32),
            jax.ShapeDtypeStruct((n_pad, LANES), jnp.float32),
        ],
        mesh=mesh,
        compiler_params=pltpu.CompilerParams(use_tc_tiling_on_sc=False),
        scratch_types=[
            pltpu.VMEM((CHUNK, hdim), jnp.float32),
            pltpu.VMEM((CHUNK, hdim), jnp.float32),
            pltpu.VMEM((CHUNK, hdim), jnp.float32),
            pltpu.VMEM((CHUNK, LANES), jnp.float32),
            pltpu.VMEM((CHUNK,), jnp.int32),
            pltpu.VMEM((CHUNK,), jnp.int32),
            pltpu.VMEM((CHUNK,), jnp.int32),
            pltpu.VMEM_SHARED((acc_rows, hdim), jnp.float32),
            pltpu.VMEM_SHARED((acc_rows, LANES), jnp.float32),
            pltpu.SemaphoreType.DMA,
            pltpu.SemaphoreType.DMA,
            pltpu.SemaphoreType.DMA,
        ],
    )
    def attn(
        q_hbm,
        k_hbm,
        v_hbm,
        rg_hbm,
        c_hbm,
        outv,
        outd,
        qbuf,
        kbuf,
        vbuf,
        exbuf,
        rgbuf,
        cbuf,
        ldst,
        accv,
        accd,
        sem1,
        sem2,
        sem3,
    ):
        core = lax.axis_index("c")
        sub = lax.axis_index("s")
        n0 = core * n_per_sc
        nj = hdim // LANES

        zero = jnp.zeros((LANES,), jnp.float32)

        def zbody(e, c):
            for j in range(nj):
                vbuf[e, pl.ds(j * LANES, LANES)] = zero
            exbuf[e, pl.ds(0, LANES)] = zero
            return c

        lax.fori_loop(0, CHUNK, zbody, 0)

        def zslab(s, c):
            @pl.when(lax.rem(s, NT) == sub)
            def _():
                pltpu.sync_copy(vbuf, accv.at[pl.ds(s * CHUNK, CHUNK)])
                pltpu.sync_copy(exbuf, accd.at[pl.ds(s * CHUNK, CHUNK)])

            return c

        lax.fori_loop(0, nslabs, zslab, 0)
        plsc.subcore_barrier()

        perm = lax.iota(jnp.int32, LANES) ^ NH
        lane = lax.iota(jnp.int32, LANES)

        def chunk_body(ci, carry):
            eb = sub * ept + ci * CHUNK
            pltpu.sync_copy(rg_hbm.at[pl.ds(eb, CHUNK)], rgbuf)
            pltpu.sync_copy(c_hbm.at[pl.ds(eb, CHUNK)], cbuf)
            dq = pltpu.async_copy(q_hbm.at[rgbuf], qbuf, sem1)
            dk = pltpu.async_copy(k_hbm.at[cbuf], kbuf, sem2)
            dv = pltpu.async_copy(v_hbm.at[cbuf], vbuf, sem3)

            def ld_body(j, c):
                r = rgbuf[pl.ds(j * LANES, LANES)]
                loc = r - n0
                eid = eb + j * LANES + lane
                ok = (loc >= 0) & (loc < n_per_sc) & (eid < n_edges)
                ldst[pl.ds(j * LANES, LANES)] = jnp.where(ok, loc, trash)
                return c

            lax.fori_loop(0, CHUNK // LANES, ld_body, 0)
            dq.wait()
            dk.wait()
            dv.wait()

            def e_body(e, c):
                t = qbuf[e, pl.ds(0, LANES)] * kbuf[e, pl.ds(0, LANES)]
                for j in range(1, nj):
                    t = t + (
                        qbuf[e, pl.ds(j * LANES, LANES)]
                        * kbuf[e, pl.ds(j * LANES, LANES)]
                    )
                tp = lax.gather(
                    t,
                    perm[:, None],
                    dimension_numbers=lax.GatherDimensionNumbers(
                        offset_dims=(),
                        collapsed_slice_dims=(0,),
                        start_index_map=(0,),
                    ),
                    slice_sizes=(1,),
                    mode=lax.GatherScatterMode.PROMISE_IN_BOUNDS,
                )
                ex = jnp.exp(jnp.minimum(t + tp, CLAMP))
                exbuf[e, pl.ds(0, LANES)] = ex
                for j in range(nj):
                    vbuf[e, pl.ds(j * LANES, LANES)] = (
                        vbuf[e, pl.ds(j * LANES, LANES)] * ex
                    )
                return c

            lax.fori_loop(0, CHUNK, e_body, 0)
            pltpu.sync_copy(vbuf, accv.at[ldst], add=True)
            pltpu.sync_copy(exbuf, accd.at[ldst], add=True)
            return carry

        lax.fori_loop(0, cpt, chunk_body, 0)
        plsc.subcore_barrier()

        wb = sub * rows_per_tile
        pltpu.sync_copy(
            accv.at[pl.ds(wb, rows_per_tile)],
            outv.at[pl.ds(n0 + wb, rows_per_tile)],
        )
        pltpu.sync_copy(
            accd.at[pl.ds(wb, rows_per_tile)],
            outd.at[pl.ds(n0 + wb, rows_per_tile)],
        )

    return attn


def kernel(h, edge_index, weights):
    n_nodes, hdim = h.shape
    e_edges = edge_index.shape[1]
    hd = hdim // NH
    scale = hd ** -0.5
    nlayers = sum(1 for kk in weights if kk.endswith("_Wq"))

    n_pad = ((n_nodes + 511) // 512) * 512
    e_pad = -(-e_edges // (NT * CHUNK)) * (NT * CHUNK)

    row = edge_index[0]
    col = edge_index[1]
    rowg = jnp.pad(row, (0, e_pad - e_edges))
    colg = jnp.pad(col, (0, e_pad - e_edges))

    def wqkv(l):
        return jnp.concatenate(
            [
                weights[f"l{l}_Wq"] * scale,
                weights[f"l{l}_Wk"],
                weights[f"l{l}_Wv"],
            ],
            axis=1,
        )

    def bqkv(l):
        return jnp.concatenate(
            [
                weights[f"l{l}_bq"] * scale,
                weights[f"l{l}_bk"],
                weights[f"l{l}_bv"],
            ]
        )

    def fold(wo, bo, w2, b2):
        k = wo.shape[0]
        aug = jnp.concatenate([wo, bo.reshape(1, -1)], axis=0)
        aug = jnp.concatenate(
            [aug, jnp.zeros((k + 1, 1), jnp.float32).at[k, 0].set(1.0)], axis=1
        )
        rhs = jnp.concatenate([w2, b2.reshape(1, -1)], axis=0)
        out = _matmul_bias(aug, rhs, jnp.zeros((rhs.shape[1],), jnp.float32))
        return out[:k], out[k]

    sc_attn = _make_sc_attention(n_pad, hdim, e_pad, e_edges)

    hp = jnp.pad(h, ((0, n_pad - n_nodes), (0, 0)))
    q, k, v = _proj3(hp, wqkv(0), bqkv(0), hdim)
    outv, outd = sc_attn(q, k, v, rowg, colg)
    for l in range(1, nlayers):
        aw, ab = fold(weights[f"l{l-1}_Wo"], weights[f"l{l-1}_bo"], wqkv(l), bqkv(l))
        q, k, v = _norm3(outv, outd, aw, ab, hdim)
        outv, outd = sc_attn(q, k, v, rowg, colg)
    lw = nlayers - 1
    wp = jnp.pad(weights["Wp"], ((0, 0), (0, 127)))
    bp = jnp.pad(weights["bp"], (0, 127))
    fw, fb = fold(weights[f"l{lw}_Wo"], weights[f"l{lw}_bo"], wp, bp)
    res = _normf(outv, outd, fw, fb, hdim)
    return res[:n_nodes, :1]


# speedup vs baseline: 23.5671x; 1.2859x over previous
import functools

import jax
import jax.numpy as jnp
from jax import lax
from jax.experimental import pallas as pl
from jax.experimental.pallas import tpu as pltpu
from jax.experimental.pallas import tpu_sc as plsc

NH = 8
LANES = 16
CHUNK = 32
NT = 16
NSC = 2
CLAMP = 60.0


def _mm_kernel(x_ref, w_ref, b_ref, o_ref):
    o_ref[...] = (
        jnp.dot(x_ref[...], w_ref[...], preferred_element_type=jnp.float32)
        + b_ref[...]
    )


def _matmul_bias(x, w, b, bm=512):
    m, k = x.shape
    n = w.shape[1]
    mp = ((m + bm - 1) // bm) * bm
    xp = jnp.pad(x, ((0, mp - m), (0, 0)))
    out = pl.pallas_call(
        _mm_kernel,
        grid=(mp // bm,),
        in_specs=[
            pl.BlockSpec((bm, k), lambda i: (i, 0)),
            pl.BlockSpec((k, n), lambda i: (0, 0)),
            pl.BlockSpec((1, n), lambda i: (0, 0)),
        ],
        out_specs=pl.BlockSpec((bm, n), lambda i: (i, 0)),
        out_shape=jax.ShapeDtypeStruct((mp, n), jnp.float32),
    )(xp, w, b.reshape(1, n))
    return out[:m]


def _proj3_kernel(hdim, x_ref, w_ref, b_ref, q_ref, k_ref, v_ref):
    y = (
        jnp.dot(x_ref[...], w_ref[...], preferred_element_type=jnp.float32)
        + b_ref[...]
    )
    q_ref[...] = y[:, :hdim]
    k_ref[...] = y[:, hdim : 2 * hdim]
    v_ref[...] = y[:, 2 * hdim :]


def _norm3_kernel(hdim, xv_ref, xd_ref, w_ref, b_ref, q_ref, k_ref, v_ref):
    inv = 1.0 / (xd_ref[...] + 1e-9)
    xn = xv_ref[...] * jnp.tile(inv, (1, hdim // LANES))
    y = jnp.dot(xn, w_ref[...], preferred_element_type=jnp.float32) + b_ref[...]
    q_ref[...] = y[:, :hdim]
    k_ref[...] = y[:, hdim : 2 * hdim]
    v_ref[...] = y[:, 2 * hdim :]


def _normf_kernel(hdim, xv_ref, xd_ref, w_ref, b_ref, o_ref):
    inv = 1.0 / (xd_ref[...] + 1e-9)
    xn = xv_ref[...] * jnp.tile(inv, (1, hdim // LANES))
    o_ref[...] = (
        jnp.dot(xn, w_ref[...], preferred_element_type=jnp.float32) + b_ref[...]
    )


def _proj3(x, w, b, hdim, bm=512):
    m = x.shape[0]
    n = w.shape[1]
    sd = jax.ShapeDtypeStruct((m, hdim), jnp.float32)
    return pl.pallas_call(
        functools.partial(_proj3_kernel, hdim),
        grid=(m // bm,),
        in_specs=[
            pl.BlockSpec((bm, x.shape[1]), lambda i: (i, 0)),
            pl.BlockSpec((x.shape[1], n), lambda i: (0, 0)),
            pl.BlockSpec((1, n), lambda i: (0, 0)),
        ],
        out_specs=[pl.BlockSpec((bm, hdim), lambda i: (i, 0))] * 3,
        out_shape=[sd, sd, sd],
    )(x, w, b.reshape(1, n))


def _norm3(xv, xd, w, b, hdim, bm=512):
    m = xv.shape[0]
    n = w.shape[1]
    sd = jax.ShapeDtypeStruct((m, hdim), jnp.float32)
    return pl.pallas_call(
        functools.partial(_norm3_kernel, hdim),
        grid=(m // bm,),
        in_specs=[
            pl.BlockSpec((bm, hdim), lambda i: (i, 0)),
            pl.BlockSpec((bm, LANES), lambda i: (i, 0)),
            pl.BlockSpec((hdim, n), lambda i: (0, 0)),
            pl.BlockSpec((1, n), lambda i: (0, 0)),
        ],
        out_specs=[pl.BlockSpec((bm, hdim), lambda i: (i, 0))] * 3,
        out_shape=[sd, sd, sd],
    )(xv, xd, w, b.reshape(1, n))


def _normf(xv, xd, w, b, hdim, bm=512):
    m = xv.shape[0]
    n = w.shape[1]
    return pl.pallas_call(
        functools.partial(_normf_kernel, hdim),
        grid=(m // bm,),
        in_specs=[
            pl.BlockSpec((bm, hdim), lambda i: (i, 0)),
            pl.BlockSpec((bm, LANES), lambda i: (i, 0)),
            pl.BlockSpec((hdim, n), lambda i: (0, 0)),
            pl.BlockSpec((1, n), lambda i: (0, 0)),
        ],
        out_specs=pl.BlockSpec((bm, n), lambda i: (i, 0)),
        out_shape=jax.ShapeDtypeStruct((m, n), jnp.float32),
    )(xv, xd, w, b.reshape(1, n))


def _make_sc_attention(n_pad, hdim, e_pad, n_edges):
    n_per_sc = n_pad // NSC
    rows_per_tile = n_per_sc // NT
    nslabs = n_per_sc // CHUNK + 1
    acc_rows = nslabs * CHUNK
    trash = n_per_sc
    cpt = e_pad // (NT * CHUNK)
    ept = cpt * CHUNK

    mesh = plsc.VectorSubcoreMesh(core_axis_name="c", subcore_axis_name="s")

    @functools.partial(
        pl.kernel,
        out_type=[
            jax.ShapeDtypeStruct((n_pad, hdim), jnp.float32),
            jax.ShapeDtypeStruct((n_pad, LANES), jnp.float32),
        ],
        mesh=mesh,
        compiler_params=pltpu.CompilerParams(use_tc_tiling_on_sc=False),
        scratch_types=[
            pltpu.VMEM((CHUNK, hdim), jnp.float32),
            pltpu.VMEM((CHUNK, hdim), jnp.float32),
            pltpu.VMEM((CHUNK, hdim), jnp.float32),
            pltpu.VMEM((CHUNK, hdim), jnp.float32),
            pltpu.VMEM((CHUNK, hdim), jnp.float32),
            pltpu.VMEM((CHUNK, LANES), jnp.float32),
            pltpu.VMEM((CHUNK, LANES), jnp.float32),
            pltpu.VMEM((2 * CHUNK,), jnp.int32),
            pltpu.VMEM((2 * CHUNK,), jnp.int32),
            pltpu.VMEM((CHUNK,), jnp.int32),
            pltpu.VMEM((CHUNK,), jnp.int32),
            pltpu.VMEM_SHARED((acc_rows, hdim), jnp.float32),
            pltpu.VMEM_SHARED((acc_rows, LANES), jnp.float32),
            pltpu.SemaphoreType.DMA,
            pltpu.SemaphoreType.DMA,
            pltpu.SemaphoreType.DMA,
            pltpu.SemaphoreType.DMA,
            pltpu.SemaphoreType.DMA,
            pltpu.SemaphoreType.DMA,
        ],
    )
    def attn(
        q_hbm,
        k_hbm,
        v_hbm,
        rc_hbm,
        outv,
        outd,
        qbufA,
        qbufB,
        kbufA,
        kbufB,
        vbuf,
        exbufA,
        exbufB,
        rcA,
        rcB,
        ldstA,
        ldstB,
        accv,
        accd,
        sqA,
        skA,
        sqB,
        skB,
        sv,
        ssc,
    ):
        core = lax.axis_index("c")
        sub = lax.axis_index("s")
        n0 = core * n_per_sc
        nj = hdim // LANES
        qb = (qbufA, qbufB)
        kb = (kbufA, kbufB)
        exb = (exbufA, exbufB)
        rc = (rcA, rcB)
        ld = (ldstA, ldstB)
        sq = (sqA, sqB)
        sk = (skA, skB)

        zero = jnp.zeros((LANES,), jnp.float32)

        def zbody(e, c):
            for j in range(nj):
                vbuf[e, pl.ds(j * LANES, LANES)] = zero
            exbufA[e, pl.ds(0, LANES)] = zero
            return c

        lax.fori_loop(0, CHUNK, zbody, 0)

        def zslab(s, c):
            @pl.when(lax.rem(s, NT) == sub)
            def _():
                pltpu.sync_copy(vbuf, accv.at[pl.ds(s * CHUNK, CHUNK)])
                pltpu.sync_copy(exbufA, accd.at[pl.ds(s * CHUNK, CHUNK)])

            return c

        lax.fori_loop(0, nslabs, zslab, 0)
        plsc.subcore_barrier()

        perm = lax.iota(jnp.int32, LANES) ^ NH
        lane = lax.iota(jnp.int32, LANES)

        def load_rc(ci, p):
            pltpu.sync_copy(
                rc_hbm.at[pl.ds((sub * cpt + ci) * 2 * CHUNK, 2 * CHUNK)], rc[p]
            )

        def comp_ld(ci, p):
            eb = sub * ept + ci * CHUNK
            for j in range(CHUNK // LANES):
                r = rc[p][pl.ds(j * LANES, LANES)]
                loc = r - n0
                eid = eb + j * LANES + lane
                ok = (loc >= 0) & (loc < n_per_sc) & (eid < n_edges)
                ld[p][pl.ds(j * LANES, LANES)] = jnp.where(ok, loc, trash)

        def fire_qk(p):
            pltpu.async_copy(q_hbm.at[rc[p].at[pl.ds(0, CHUNK)]], qb[p], sq[p])
            pltpu.async_copy(
                k_hbm.at[rc[p].at[pl.ds(CHUNK, CHUNK)]], kb[p], sk[p]
            )

        def wait_qk(p):
            pltpu.make_async_copy(q_hbm.at[pl.ds(0, CHUNK)], qb[p], sq[p]).wait()
            pltpu.make_async_copy(k_hbm.at[pl.ds(0, CHUNK)], kb[p], sk[p]).wait()

        def wait_scatter(p):
            pltpu.make_async_copy(vbuf, accv.at[ld[p]], ssc).wait()
            pltpu.make_async_copy(exb[p], accd.at[ld[p]], ssc).wait()

        def do_chunk(ci, p, first, last):
            po = 1 - p
            if not last:
                fire_qk(po)
            if not first:
                wait_scatter(po)
                comp_ld(ci + 1, po)
            dv = pltpu.async_copy(v_hbm.at[rc[p].at[pl.ds(CHUNK, CHUNK)]], vbuf, sv)
            wait_qk(p)

            def score_body(e, c):
                t0 = qb[p][e, pl.ds(0, LANES)] * kb[p][e, pl.ds(0, LANES)]
                t1 = qb[p][e, pl.ds(LANES, LANES)] * kb[p][e, pl.ds(LANES, LANES)]
                for j in range(2, nj, 2):
                    t0 = t0 + (
                        qb[p][e, pl.ds(j * LANES, LANES)]
                        * kb[p][e, pl.ds(j * LANES, LANES)]
                    )
                    t1 = t1 + (
                        qb[p][e, pl.ds((j + 1) * LANES, LANES)]
                        * kb[p][e, pl.ds((j + 1) * LANES, LANES)]
                    )
                t = t0 + t1
                tp = lax.gather(
                    t,
                    perm[:, None],
                    dimension_numbers=lax.GatherDimensionNumbers(
                        offset_dims=(),
                        collapsed_slice_dims=(0,),
                        start_index_map=(0,),
                    ),
                    slice_sizes=(1,),
                    mode=lax.GatherScatterMode.PROMISE_IN_BOUNDS,
                )
                ex = jnp.exp(jnp.minimum(t + tp, CLAMP))
                exb[p][e, pl.ds(0, LANES)] = ex
                return c

            lax.fori_loop(0, CHUNK, score_body, 0)
            dv.wait()

            def vmul_body(e, c):
                ex = exb[p][e, pl.ds(0, LANES)]
                for j in range(nj):
                    vbuf[e, pl.ds(j * LANES, LANES)] = (
                        vbuf[e, pl.ds(j * LANES, LANES)] * ex
                    )
                return c

            lax.fori_loop(0, CHUNK, vmul_body, 0)
            pltpu.async_copy(vbuf, accv.at[ld[p]], ssc, add=True)
            pltpu.async_copy(exb[p], accd.at[ld[p]], ssc, add=True)
            if not last:
                @pl.when(ci + 2 < cpt)
                def _():
                    load_rc(ci + 2, p)

        load_rc(0, 0)
        comp_ld(0, 0)
        load_rc(1, 1)
        comp_ld(1, 1)
        fire_qk(0)

        def pair_body(i, carry):
            c0 = 2 * i

            @pl.when(c0 == 0)
            def _():
                do_chunk(c0, 0, True, False)

            @pl.when(c0 > 0)
            def _():
                do_chunk(c0, 0, False, False)

            @pl.when(c0 + 1 == cpt - 1)
            def _():
                do_chunk(c0 + 1, 1, False, True)

            @pl.when(c0 + 1 < cpt - 1)
            def _():
                do_chunk(c0 + 1, 1, False, False)

            return carry

        lax.fori_loop(0, cpt // 2, pair_body, 0)
        wait_scatter(1)
        plsc.subcore_barrier()

        wb = sub * rows_per_tile
        pltpu.sync_copy(
            accv.at[pl.ds(wb, rows_per_tile)],
            outv.at[pl.ds(n0 + wb, rows_per_tile)],
        )
        pltpu.sync_copy(
            accd.at[pl.ds(wb, rows_per_tile)],
            outd.at[pl.ds(n0 + wb, rows_per_tile)],
        )

    return attn


def kernel(h, edge_index, weights):
    n_nodes, hdim = h.shape
    e_edges = edge_index.shape[1]
    hd = hdim // NH
    scale = hd ** -0.5
    nlayers = sum(1 for kk in weights if kk.endswith("_Wq"))

    n_pad = ((n_nodes + 511) // 512) * 512
    cpt = -(-e_edges // (NT * CHUNK))
    cpt = cpt + (cpt % 2)
    e_pad = cpt * NT * CHUNK

    row = edge_index[0]
    col = edge_index[1]
    rowg = jnp.pad(row, (0, e_pad - e_edges))
    colg = jnp.pad(col, (0, e_pad - e_edges))
    rc = jnp.concatenate(
        [rowg.reshape(-1, CHUNK), colg.reshape(-1, CHUNK)], axis=1
    ).reshape(-1)

    def wqkv(l):
        return jnp.concatenate(
            [
                weights[f"l{l}_Wq"] * scale,
                weights[f"l{l}_Wk"],
                weights[f"l{l}_Wv"],
            ],
            axis=1,
        )

    def bqkv(l):
        return jnp.concatenate(
            [
                weights[f"l{l}_bq"] * scale,
                weights[f"l{l}_bk"],
                weights[f"l{l}_bv"],
            ]
        )

    def fold(wo, bo, w2, b2):
        k = wo.shape[0]
        aug = jnp.concatenate([wo, bo.reshape(1, -1)], axis=0)
        aug = jnp.concatenate(
            [aug, jnp.zeros((k + 1, 1), jnp.float32).at[k, 0].set(1.0)], axis=1
        )
        rhs = jnp.concatenate([w2, b2.reshape(1, -1)], axis=0)
        out = _matmul_bias(aug, rhs, jnp.zeros((rhs.shape[1],), jnp.float32))
        return out[:k], out[k]

    sc_attn = _make_sc_attention(n_pad, hdim, e_pad, e_edges)

    hp = jnp.pad(h, ((0, n_pad - n_nodes), (0, 0)))
    q, k, v = _proj3(hp, wqkv(0), bqkv(0), hdim)
    outv, outd = sc_attn(q, k, v, rc)
    for l in range(1, nlayers):
        aw, ab = fold(weights[f"l{l-1}_Wo"], weights[f"l{l-1}_bo"], wqkv(l), bqkv(l))
        q, k, v = _norm3(outv, outd, aw, ab, hdim)
        outv, outd = sc_attn(q, k, v, rc)
    lw = nlayers - 1
    wp = jnp.pad(weights["Wp"], ((0, 0), (0, 127)))
    bp = jnp.pad(weights["bp"], (0, 127))
    fw, fb = fold(weights[f"l{lw}_Wo"], weights[f"l{lw}_bo"], wp, bp)
    res = _normf(outv, outd, fw, fb, hdim)
    return res[:n_nodes, :1]


# speedup vs baseline: 24.7022x; 1.0482x over previous
import functools

import jax
import jax.numpy as jnp
from jax import lax
from jax.experimental import pallas as pl
from jax.experimental.pallas import tpu as pltpu
from jax.experimental.pallas import tpu_sc as plsc

NH = 8
LANES = 16
CHUNK = 32
NT = 16
NSC = 2
CLAMP = 60.0


def _mm_kernel(x_ref, w_ref, b_ref, o_ref):
    o_ref[...] = (
        jnp.dot(x_ref[...], w_ref[...], preferred_element_type=jnp.float32)
        + b_ref[...]
    )


def _matmul_bias(x, w, b, bm=512):
    m, k = x.shape
    n = w.shape[1]
    mp = ((m + bm - 1) // bm) * bm
    xp = jnp.pad(x, ((0, mp - m), (0, 0)))
    out = pl.pallas_call(
        _mm_kernel,
        grid=(mp // bm,),
        in_specs=[
            pl.BlockSpec((bm, k), lambda i: (i, 0)),
            pl.BlockSpec((k, n), lambda i: (0, 0)),
            pl.BlockSpec((1, n), lambda i: (0, 0)),
        ],
        out_specs=pl.BlockSpec((bm, n), lambda i: (i, 0)),
        out_shape=jax.ShapeDtypeStruct((mp, n), jnp.float32),
    )(xp, w, b.reshape(1, n))
    return out[:m]


def _proj3_kernel(hdim, x_ref, w_ref, b_ref, q_ref, k_ref, v_ref):
    y = (
        jnp.dot(x_ref[...], w_ref[...], preferred_element_type=jnp.float32)
        + b_ref[...]
    )
    q_ref[...] = y[:, :hdim]
    k_ref[...] = y[:, hdim : 2 * hdim]
    v_ref[...] = y[:, 2 * hdim :]


def _norm3_kernel(hdim, xv_ref, xd_ref, w_ref, b_ref, q_ref, k_ref, v_ref):
    inv = 1.0 / (xd_ref[...] + 1e-9)
    xn = xv_ref[...] * jnp.tile(inv, (1, hdim // LANES))
    y = jnp.dot(xn, w_ref[...], preferred_element_type=jnp.float32) + b_ref[...]
    q_ref[...] = y[:, :hdim]
    k_ref[...] = y[:, hdim : 2 * hdim]
    v_ref[...] = y[:, 2 * hdim :]


def _normf_kernel(hdim, xv_ref, xd_ref, w_ref, b_ref, o_ref):
    inv = 1.0 / (xd_ref[...] + 1e-9)
    xn = xv_ref[...] * jnp.tile(inv, (1, hdim // LANES))
    o_ref[...] = (
        jnp.dot(xn, w_ref[...], preferred_element_type=jnp.float32) + b_ref[...]
    )


def _proj3(x, w, b, hdim, bm=512):
    m = x.shape[0]
    n = w.shape[1]
    sd = jax.ShapeDtypeStruct((m, hdim), jnp.float32)
    return pl.pallas_call(
        functools.partial(_proj3_kernel, hdim),
        grid=(m // bm,),
        in_specs=[
            pl.BlockSpec((bm, x.shape[1]), lambda i: (i, 0)),
            pl.BlockSpec((x.shape[1], n), lambda i: (0, 0)),
            pl.BlockSpec((1, n), lambda i: (0, 0)),
        ],
        out_specs=[pl.BlockSpec((bm, hdim), lambda i: (i, 0))] * 3,
        out_shape=[sd, sd, sd],
    )(x, w, b.reshape(1, n))


def _norm3(xv, xd, w, b, hdim, bm=512):
    m = xv.shape[0]
    n = w.shape[1]
    sd = jax.ShapeDtypeStruct((m, hdim), jnp.float32)
    return pl.pallas_call(
        functools.partial(_norm3_kernel, hdim),
        grid=(m // bm,),
        in_specs=[
            pl.BlockSpec((bm, hdim), lambda i: (i, 0)),
            pl.BlockSpec((bm, LANES), lambda i: (i, 0)),
            pl.BlockSpec((hdim, n), lambda i: (0, 0)),
            pl.BlockSpec((1, n), lambda i: (0, 0)),
        ],
        out_specs=[pl.BlockSpec((bm, hdim), lambda i: (i, 0))] * 3,
        out_shape=[sd, sd, sd],
    )(xv, xd, w, b.reshape(1, n))


def _normf(xv, xd, w, b, hdim, bm=512):
    m = xv.shape[0]
    n = w.shape[1]
    return pl.pallas_call(
        functools.partial(_normf_kernel, hdim),
        grid=(m // bm,),
        in_specs=[
            pl.BlockSpec((bm, hdim), lambda i: (i, 0)),
            pl.BlockSpec((bm, LANES), lambda i: (i, 0)),
            pl.BlockSpec((hdim, n), lambda i: (0, 0)),
            pl.BlockSpec((1, n), lambda i: (0, 0)),
        ],
        out_specs=pl.BlockSpec((bm, n), lambda i: (i, 0)),
        out_shape=jax.ShapeDtypeStruct((m, n), jnp.float32),
    )(xv, xd, w, b.reshape(1, n))


def _make_sc_attention(n_pad, hdim, e_pad, n_edges):
    n_per_sc = n_pad // NSC
    rows_per_tile = n_per_sc // NT
    nslabs = n_per_sc // CHUNK + 1
    acc_rows = nslabs * CHUNK
    trash = n_per_sc
    cpt = e_pad // (NT * CHUNK)
    ept = cpt * CHUNK

    mesh = plsc.VectorSubcoreMesh(core_axis_name="c", subcore_axis_name="s")

    @functools.partial(
        pl.kernel,
        out_type=[
            jax.ShapeDtypeStruct((n_pad, hdim), jnp.float32),
            jax.ShapeDtypeStruct((n_pad, LANES), jnp.float32),
        ],
        mesh=mesh,
        compiler_params=pltpu.CompilerParams(use_tc_tiling_on_sc=False),
        scratch_types=[
            pltpu.VMEM((CHUNK, hdim), jnp.float32),
            pltpu.VMEM((CHUNK, hdim), jnp.float32),
            pltpu.VMEM((CHUNK, hdim), jnp.float32),
            pltpu.VMEM((CHUNK, hdim), jnp.float32),
            pltpu.VMEM((CHUNK, hdim), jnp.float32),
            pltpu.VMEM((CHUNK, LANES), jnp.float32),
            pltpu.VMEM((CHUNK, LANES), jnp.float32),
            pltpu.VMEM((2 * CHUNK,), jnp.int32),
            pltpu.VMEM((2 * CHUNK,), jnp.int32),
            pltpu.VMEM((CHUNK,), jnp.int32),
            pltpu.VMEM((CHUNK,), jnp.int32),
            pltpu.VMEM_SHARED((acc_rows, hdim), jnp.float32),
            pltpu.VMEM_SHARED((acc_rows, LANES), jnp.float32),
            pltpu.SemaphoreType.DMA,
            pltpu.SemaphoreType.DMA,
            pltpu.SemaphoreType.DMA,
            pltpu.SemaphoreType.DMA,
            pltpu.SemaphoreType.DMA,
            pltpu.SemaphoreType.DMA,
        ],
    )
    def attn(
        q_hbm,
        k_hbm,
        v_hbm,
        rc_hbm,
        outv,
        outd,
        qbufA,
        qbufB,
        kbufA,
        kbufB,
        vbuf,
        exbufA,
        exbufB,
        rcA,
        rcB,
        ldstA,
        ldstB,
        accv,
        accd,
        sqA,
        skA,
        sqB,
        skB,
        sv,
        ssc,
    ):
        core = lax.axis_index("c")
        sub = lax.axis_index("s")
        n0 = core * n_per_sc
        nj = hdim // LANES
        qb = (qbufA, qbufB)
        kb = (kbufA, kbufB)
        exb = (exbufA, exbufB)
        rc = (rcA, rcB)
        ld = (ldstA, ldstB)
        sq = (sqA, sqB)
        sk = (skA, skB)

        zero = jnp.zeros((LANES,), jnp.float32)

        def zbody(e, c):
            for j in range(nj):
                vbuf[e, pl.ds(j * LANES, LANES)] = zero
            exbufA[e, pl.ds(0, LANES)] = zero
            return c

        lax.fori_loop(0, CHUNK, zbody, 0)

        def zslab(s, c):
            @pl.when(lax.rem(s, NT) == sub)
            def _():
                pltpu.sync_copy(vbuf, accv.at[pl.ds(s * CHUNK, CHUNK)])
                pltpu.sync_copy(exbufA, accd.at[pl.ds(s * CHUNK, CHUNK)])

            return c

        lax.fori_loop(0, nslabs, zslab, 0)
        plsc.subcore_barrier()

        perm = lax.iota(jnp.int32, LANES) ^ NH
        lane = lax.iota(jnp.int32, LANES)

        def load_rc(ci, p):
            pltpu.sync_copy(
                rc_hbm.at[pl.ds((sub * cpt + ci) * 2 * CHUNK, 2 * CHUNK)], rc[p]
            )

        def comp_ld(ci, p):
            eb = sub * ept + ci * CHUNK
            for j in range(CHUNK // LANES):
                r = rc[p][pl.ds(j * LANES, LANES)]
                loc = r - n0
                eid = eb + j * LANES + lane
                ok = (loc >= 0) & (loc < n_per_sc) & (eid < n_edges)
                ld[p][pl.ds(j * LANES, LANES)] = jnp.where(ok, loc, trash)

        def fire_qk(p):
            pltpu.async_copy(q_hbm.at[rc[p].at[pl.ds(0, CHUNK)]], qb[p], sq[p])
            pltpu.async_copy(
                k_hbm.at[rc[p].at[pl.ds(CHUNK, CHUNK)]], kb[p], sk[p]
            )

        def wait_qk(p):
            pltpu.make_async_copy(q_hbm.at[pl.ds(0, CHUNK)], qb[p], sq[p]).wait()
            pltpu.make_async_copy(k_hbm.at[pl.ds(0, CHUNK)], kb[p], sk[p]).wait()

        def wait_scatter(p):
            pltpu.make_async_copy(vbuf, accv.at[ld[p]], ssc).wait()
            pltpu.make_async_copy(exb[p], accd.at[ld[p]], ssc).wait()

        def do_chunk(ci, p, first, last):
            po = 1 - p
            if not last:
                fire_qk(po)
            if not first:
                wait_scatter(po)
                comp_ld(ci + 1, po)
            dv = pltpu.async_copy(v_hbm.at[rc[p].at[pl.ds(CHUNK, CHUNK)]], vbuf, sv)
            wait_qk(p)

            @plsc.parallel_loop(0, CHUNK, 1, unroll=4)
            def score_body(e):
                t0 = qb[p][e, pl.ds(0, LANES)] * kb[p][e, pl.ds(0, LANES)]
                t1 = qb[p][e, pl.ds(LANES, LANES)] * kb[p][e, pl.ds(LANES, LANES)]
                for j in range(2, nj, 2):
                    t0 = t0 + (
                        qb[p][e, pl.ds(j * LANES, LANES)]
                        * kb[p][e, pl.ds(j * LANES, LANES)]
                    )
                    t1 = t1 + (
                        qb[p][e, pl.ds((j + 1) * LANES, LANES)]
                        * kb[p][e, pl.ds((j + 1) * LANES, LANES)]
                    )
                t = t0 + t1
                tp = lax.gather(
                    t,
                    perm[:, None],
                    dimension_numbers=lax.GatherDimensionNumbers(
                        offset_dims=(),
                        collapsed_slice_dims=(0,),
                        start_index_map=(0,),
                    ),
                    slice_sizes=(1,),
                    mode=lax.GatherScatterMode.PROMISE_IN_BOUNDS,
                )
                ex = jnp.exp(jnp.minimum(t + tp, CLAMP))
                exb[p][e, pl.ds(0, LANES)] = ex

            dv.wait()

            @plsc.parallel_loop(0, CHUNK, 1, unroll=4)
            def vmul_body(e):
                ex = exb[p][e, pl.ds(0, LANES)]
                for j in range(nj):
                    vbuf[e, pl.ds(j * LANES, LANES)] = (
                        vbuf[e, pl.ds(j * LANES, LANES)] * ex
                    )
            pltpu.async_copy(vbuf, accv.at[ld[p]], ssc, add=True)
            pltpu.async_copy(exb[p], accd.at[ld[p]], ssc, add=True)
            if not last:
                @pl.when(ci + 2 < cpt)
                def _():
                    load_rc(ci + 2, p)

        load_rc(0, 0)
        comp_ld(0, 0)
        load_rc(1, 1)
        comp_ld(1, 1)
        fire_qk(0)

        def pair_body(i, carry):
            c0 = 2 * i

            @pl.when(c0 == 0)
            def _():
                do_chunk(c0, 0, True, False)

            @pl.when(c0 > 0)
            def _():
                do_chunk(c0, 0, False, False)

            @pl.when(c0 + 1 == cpt - 1)
            def _():
                do_chunk(c0 + 1, 1, False, True)

            @pl.when(c0 + 1 < cpt - 1)
            def _():
                do_chunk(c0 + 1, 1, False, False)

            return carry

        lax.fori_loop(0, cpt // 2, pair_body, 0)
        wait_scatter(1)
        plsc.subcore_barrier()

        wb = sub * rows_per_tile
        pltpu.sync_copy(
            accv.at[pl.ds(wb, rows_per_tile)],
            outv.at[pl.ds(n0 + wb, rows_per_tile)],
        )
        pltpu.sync_copy(
            accd.at[pl.ds(wb, rows_per_tile)],
            outd.at[pl.ds(n0 + wb, rows_per_tile)],
        )

    return attn


def kernel(h, edge_index, weights):
    n_nodes, hdim = h.shape
    e_edges = edge_index.shape[1]
    hd = hdim // NH
    scale = hd ** -0.5
    nlayers = sum(1 for kk in weights if kk.endswith("_Wq"))

    n_pad = ((n_nodes + 511) // 512) * 512
    cpt = -(-e_edges // (NT * CHUNK))
    cpt = cpt + (cpt % 2)
    e_pad = cpt * NT * CHUNK

    row = edge_index[0]
    col = edge_index[1]
    rowg = jnp.pad(row, (0, e_pad - e_edges))
    colg = jnp.pad(col, (0, e_pad - e_edges))
    rc = jnp.concatenate(
        [rowg.reshape(-1, CHUNK), colg.reshape(-1, CHUNK)], axis=1
    ).reshape(-1)

    def wqkv(l):
        return jnp.concatenate(
            [
                weights[f"l{l}_Wq"] * scale,
                weights[f"l{l}_Wk"],
                weights[f"l{l}_Wv"],
            ],
            axis=1,
        )

    def bqkv(l):
        return jnp.concatenate(
            [
                weights[f"l{l}_bq"] * scale,
                weights[f"l{l}_bk"],
                weights[f"l{l}_bv"],
            ]
        )

    def fold(wo, bo, w2, b2):
        k = wo.shape[0]
        aug = jnp.concatenate([wo, bo.reshape(1, -1)], axis=0)
        aug = jnp.concatenate(
            [aug, jnp.zeros((k + 1, 1), jnp.float32).at[k, 0].set(1.0)], axis=1
        )
        rhs = jnp.concatenate([w2, b2.reshape(1, -1)], axis=0)
        out = _matmul_bias(aug, rhs, jnp.zeros((rhs.shape[1],), jnp.float32))
        return out[:k], out[k]

    sc_attn = _make_sc_attention(n_pad, hdim, e_pad, e_edges)

    hp = jnp.pad(h, ((0, n_pad - n_nodes), (0, 0)))
    q, k, v = _proj3(hp, wqkv(0), bqkv(0), hdim)
    outv, outd = sc_attn(q, k, v, rc)
    for l in range(1, nlayers):
        aw, ab = fold(weights[f"l{l-1}_Wo"], weights[f"l{l-1}_bo"], wqkv(l), bqkv(l))
        q, k, v = _norm3(outv, outd, aw, ab, hdim)
        outv, outd = sc_attn(q, k, v, rc)
    lw = nlayers - 1
    wp = jnp.pad(weights["Wp"], ((0, 0), (0, 127)))
    bp = jnp.pad(weights["bp"], (0, 127))
    fw, fb = fold(weights[f"l{lw}_Wo"], weights[f"l{lw}_bo"], wp, bp)
    res = _normf(outv, outd, fw, fb, hdim)
    return res[:n_nodes, :1]


# speedup vs baseline: 24.7284x; 1.0011x over previous
import functools

import jax
import jax.numpy as jnp
from jax import lax
from jax.experimental import pallas as pl
from jax.experimental.pallas import tpu as pltpu
from jax.experimental.pallas import tpu_sc as plsc

NH = 8
LANES = 16
CHUNK = 32
NT = 16
NSC = 2
CLAMP = 60.0


def _mm_kernel(x_ref, w_ref, b_ref, o_ref):
    o_ref[...] = (
        jnp.dot(x_ref[...], w_ref[...], preferred_element_type=jnp.float32)
        + b_ref[...]
    )


def _matmul_bias(x, w, b, bm=512):
    m, k = x.shape
    n = w.shape[1]
    mp = ((m + bm - 1) // bm) * bm
    xp = jnp.pad(x, ((0, mp - m), (0, 0)))
    out = pl.pallas_call(
        _mm_kernel,
        grid=(mp // bm,),
        in_specs=[
            pl.BlockSpec((bm, k), lambda i: (i, 0)),
            pl.BlockSpec((k, n), lambda i: (0, 0)),
            pl.BlockSpec((1, n), lambda i: (0, 0)),
        ],
        out_specs=pl.BlockSpec((bm, n), lambda i: (i, 0)),
        out_shape=jax.ShapeDtypeStruct((mp, n), jnp.float32),
    )(xp, w, b.reshape(1, n))
    return out[:m]


def _proj3_kernel(hdim, x_ref, w_ref, b_ref, q_ref, k_ref, v_ref):
    y = (
        jnp.dot(x_ref[...], w_ref[...], preferred_element_type=jnp.float32)
        + b_ref[...]
    )
    q_ref[...] = y[:, :hdim]
    k_ref[...] = y[:, hdim : 2 * hdim]
    v_ref[...] = y[:, 2 * hdim :]


def _norm3_kernel(hdim, xv_ref, xd_ref, w_ref, b_ref, q_ref, k_ref, v_ref):
    inv = 1.0 / (xd_ref[...] + 1e-9)
    xn = xv_ref[...] * jnp.tile(inv, (1, hdim // LANES))
    y = jnp.dot(xn, w_ref[...], preferred_element_type=jnp.float32) + b_ref[...]
    q_ref[...] = y[:, :hdim]
    k_ref[...] = y[:, hdim : 2 * hdim]
    v_ref[...] = y[:, 2 * hdim :]


def _normf_kernel(hdim, xv_ref, xd_ref, w_ref, b_ref, o_ref):
    inv = 1.0 / (xd_ref[...] + 1e-9)
    xn = xv_ref[...] * jnp.tile(inv, (1, hdim // LANES))
    o_ref[...] = (
        jnp.dot(xn, w_ref[...], preferred_element_type=jnp.float32) + b_ref[...]
    )


def _proj3(x, w, b, hdim, bm=512):
    m = x.shape[0]
    n = w.shape[1]
    sd = jax.ShapeDtypeStruct((m, hdim), jnp.float32)
    return pl.pallas_call(
        functools.partial(_proj3_kernel, hdim),
        grid=(m // bm,),
        in_specs=[
            pl.BlockSpec((bm, x.shape[1]), lambda i: (i, 0)),
            pl.BlockSpec((x.shape[1], n), lambda i: (0, 0)),
            pl.BlockSpec((1, n), lambda i: (0, 0)),
        ],
        out_specs=[pl.BlockSpec((bm, hdim), lambda i: (i, 0))] * 3,
        out_shape=[sd, sd, sd],
    )(x, w, b.reshape(1, n))


def _norm3(xv, xd, w, b, hdim, bm=512):
    m = xv.shape[0]
    n = w.shape[1]
    sd = jax.ShapeDtypeStruct((m, hdim), jnp.float32)
    return pl.pallas_call(
        functools.partial(_norm3_kernel, hdim),
        grid=(m // bm,),
        in_specs=[
            pl.BlockSpec((bm, hdim), lambda i: (i, 0)),
            pl.BlockSpec((bm, LANES), lambda i: (i, 0)),
            pl.BlockSpec((hdim, n), lambda i: (0, 0)),
            pl.BlockSpec((1, n), lambda i: (0, 0)),
        ],
        out_specs=[pl.BlockSpec((bm, hdim), lambda i: (i, 0))] * 3,
        out_shape=[sd, sd, sd],
    )(xv, xd, w, b.reshape(1, n))


def _normf(xv, xd, w, b, hdim, bm=512):
    m = xv.shape[0]
    n = w.shape[1]
    return pl.pallas_call(
        functools.partial(_normf_kernel, hdim),
        grid=(m // bm,),
        in_specs=[
            pl.BlockSpec((bm, hdim), lambda i: (i, 0)),
            pl.BlockSpec((bm, LANES), lambda i: (i, 0)),
            pl.BlockSpec((hdim, n), lambda i: (0, 0)),
            pl.BlockSpec((1, n), lambda i: (0, 0)),
        ],
        out_specs=pl.BlockSpec((bm, n), lambda i: (i, 0)),
        out_shape=jax.ShapeDtypeStruct((m, n), jnp.float32),
    )(xv, xd, w, b.reshape(1, n))


def _make_sc_attention(n_pad, hdim, e_pad, n_edges):
    n_per_sc = n_pad // NSC
    rows_per_tile = n_per_sc // NT
    nslabs = n_per_sc // CHUNK + 1
    acc_rows = nslabs * CHUNK
    trash = n_per_sc
    cpt = e_pad // (NT * CHUNK)
    ept = cpt * CHUNK

    mesh = plsc.VectorSubcoreMesh(core_axis_name="c", subcore_axis_name="s")

    @functools.partial(
        pl.kernel,
        out_type=[
            jax.ShapeDtypeStruct((n_pad, hdim), jnp.float32),
            jax.ShapeDtypeStruct((n_pad, LANES), jnp.float32),
        ],
        mesh=mesh,
        compiler_params=pltpu.CompilerParams(use_tc_tiling_on_sc=False),
        scratch_types=[
            pltpu.VMEM((CHUNK, hdim), jnp.float32),
            pltpu.VMEM((CHUNK, hdim), jnp.float32),
            pltpu.VMEM((CHUNK, hdim), jnp.float32),
            pltpu.VMEM((CHUNK, hdim), jnp.float32),
            pltpu.VMEM((CHUNK, hdim), jnp.float32),
            pltpu.VMEM((CHUNK, LANES), jnp.float32),
            pltpu.VMEM((CHUNK, LANES), jnp.float32),
            pltpu.VMEM((2 * CHUNK,), jnp.int32),
            pltpu.VMEM((2 * CHUNK,), jnp.int32),
            pltpu.VMEM((CHUNK,), jnp.int32),
            pltpu.VMEM((CHUNK,), jnp.int32),
            pltpu.VMEM_SHARED((acc_rows, hdim), jnp.float32),
            pltpu.VMEM_SHARED((acc_rows, LANES), jnp.float32),
            pltpu.SemaphoreType.DMA,
            pltpu.SemaphoreType.DMA,
            pltpu.SemaphoreType.DMA,
            pltpu.SemaphoreType.DMA,
            pltpu.SemaphoreType.DMA,
            pltpu.SemaphoreType.DMA,
            pltpu.SemaphoreType.DMA,
            pltpu.SemaphoreType.DMA,
        ],
    )
    def attn(
        q_hbm,
        k_hbm,
        v_hbm,
        rc_hbm,
        outv,
        outd,
        qbufA,
        qbufB,
        kbufA,
        kbufB,
        vbuf,
        exbufA,
        exbufB,
        rcA,
        rcB,
        ldstA,
        ldstB,
        accv,
        accd,
        sqA,
        skA,
        sqB,
        skB,
        sv,
        ssc,
        srcA,
        srcB,
    ):
        core = lax.axis_index("c")
        sub = lax.axis_index("s")
        n0 = core * n_per_sc
        nj = hdim // LANES
        qb = (qbufA, qbufB)
        kb = (kbufA, kbufB)
        exb = (exbufA, exbufB)
        rc = (rcA, rcB)
        ld = (ldstA, ldstB)
        sq = (sqA, sqB)
        sk = (skA, skB)
        src = (srcA, srcB)

        zero = jnp.zeros((LANES,), jnp.float32)

        def zbody(e, c):
            for j in range(nj):
                vbuf[e, pl.ds(j * LANES, LANES)] = zero
            exbufA[e, pl.ds(0, LANES)] = zero
            return c

        lax.fori_loop(0, CHUNK, zbody, 0)

        def zslab(s, c):
            @pl.when(lax.rem(s, NT) == sub)
            def _():
                pltpu.sync_copy(vbuf, accv.at[pl.ds(s * CHUNK, CHUNK)])
                pltpu.sync_copy(exbufA, accd.at[pl.ds(s * CHUNK, CHUNK)])

            return c

        lax.fori_loop(0, nslabs, zslab, 0)
        plsc.subcore_barrier()

        perm = lax.iota(jnp.int32, LANES) ^ NH
        lane = lax.iota(jnp.int32, LANES)

        def load_rc(ci, p):
            pltpu.async_copy(
                rc_hbm.at[pl.ds((sub * cpt + ci) * 2 * CHUNK, 2 * CHUNK)],
                rc[p],
                src[p],
            )

        def wait_rc(p):
            pltpu.make_async_copy(
                rc_hbm.at[pl.ds(0, 2 * CHUNK)], rc[p], src[p]
            ).wait()

        def comp_ld(ci, p):
            eb = sub * ept + ci * CHUNK
            for j in range(CHUNK // LANES):
                r = rc[p][pl.ds(j * LANES, LANES)]
                loc = r - n0
                eid = eb + j * LANES + lane
                ok = (loc >= 0) & (loc < n_per_sc) & (eid < n_edges)
                ld[p][pl.ds(j * LANES, LANES)] = jnp.where(ok, loc, trash)

        def fire_qk(p):
            pltpu.async_copy(q_hbm.at[rc[p].at[pl.ds(0, CHUNK)]], qb[p], sq[p])
            pltpu.async_copy(
                k_hbm.at[rc[p].at[pl.ds(CHUNK, CHUNK)]], kb[p], sk[p]
            )

        def wait_qk(p):
            pltpu.make_async_copy(q_hbm.at[pl.ds(0, CHUNK)], qb[p], sq[p]).wait()
            pltpu.make_async_copy(k_hbm.at[pl.ds(0, CHUNK)], kb[p], sk[p]).wait()

        def wait_scatter(p):
            pltpu.make_async_copy(vbuf, accv.at[ld[p]], ssc).wait()
            pltpu.make_async_copy(exb[p], accd.at[ld[p]], ssc).wait()

        def do_chunk(ci, p, first, last):
            po = 1 - p
            if not last:
                wait_rc(po)
                fire_qk(po)
            if first:
                comp_ld(ci + 1, po)
            else:
                wait_scatter(po)
                comp_ld(ci + 1, po)
            dv = pltpu.async_copy(v_hbm.at[rc[p].at[pl.ds(CHUNK, CHUNK)]], vbuf, sv)
            wait_qk(p)

            @plsc.parallel_loop(0, CHUNK, 1, unroll=4)
            def score_body(e):
                t0 = qb[p][e, pl.ds(0, LANES)] * kb[p][e, pl.ds(0, LANES)]
                t1 = qb[p][e, pl.ds(LANES, LANES)] * kb[p][e, pl.ds(LANES, LANES)]
                for j in range(2, nj, 2):
                    t0 = t0 + (
                        qb[p][e, pl.ds(j * LANES, LANES)]
                        * kb[p][e, pl.ds(j * LANES, LANES)]
                    )
                    t1 = t1 + (
                        qb[p][e, pl.ds((j + 1) * LANES, LANES)]
                        * kb[p][e, pl.ds((j + 1) * LANES, LANES)]
                    )
                t = t0 + t1
                tp = lax.gather(
                    t,
                    perm[:, None],
                    dimension_numbers=lax.GatherDimensionNumbers(
                        offset_dims=(),
                        collapsed_slice_dims=(0,),
                        start_index_map=(0,),
                    ),
                    slice_sizes=(1,),
                    mode=lax.GatherScatterMode.PROMISE_IN_BOUNDS,
                )
                ex = jnp.exp(jnp.minimum(t + tp, CLAMP))
                exb[p][e, pl.ds(0, LANES)] = ex

            dv.wait()

            @plsc.parallel_loop(0, CHUNK, 1, unroll=4)
            def vmul_body(e):
                ex = exb[p][e, pl.ds(0, LANES)]
                for j in range(nj):
                    vbuf[e, pl.ds(j * LANES, LANES)] = (
                        vbuf[e, pl.ds(j * LANES, LANES)] * ex
                    )
            pltpu.async_copy(vbuf, accv.at[ld[p]], ssc, add=True)
            pltpu.async_copy(exb[p], accd.at[ld[p]], ssc, add=True)
            if not last:
                @pl.when(ci + 2 < cpt)
                def _():
                    load_rc(ci + 2, p)

        load_rc(0, 0)
        load_rc(1, 1)
        wait_rc(0)
        comp_ld(0, 0)
        fire_qk(0)

        def pair_body(i, carry):
            c0 = 2 * i

            @pl.when(c0 == 0)
            def _():
                do_chunk(c0, 0, True, False)

            @pl.when(c0 > 0)
            def _():
                do_chunk(c0, 0, False, False)

            @pl.when(c0 + 1 == cpt - 1)
            def _():
                do_chunk(c0 + 1, 1, False, True)

            @pl.when(c0 + 1 < cpt - 1)
            def _():
                do_chunk(c0 + 1, 1, False, False)

            return carry

        lax.fori_loop(0, cpt // 2, pair_body, 0)
        wait_scatter(1)
        plsc.subcore_barrier()

        wb = sub * rows_per_tile
        pltpu.sync_copy(
            accv.at[pl.ds(wb, rows_per_tile)],
            outv.at[pl.ds(n0 + wb, rows_per_tile)],
        )
        pltpu.sync_copy(
            accd.at[pl.ds(wb, rows_per_tile)],
            outd.at[pl.ds(n0 + wb, rows_per_tile)],
        )

    return attn


def kernel(h, edge_index, weights):
    n_nodes, hdim = h.shape
    e_edges = edge_index.shape[1]
    hd = hdim // NH
    scale = hd ** -0.5
    nlayers = sum(1 for kk in weights if kk.endswith("_Wq"))

    n_pad = ((n_nodes + 511) // 512) * 512
    cpt = -(-e_edges // (NT * CHUNK))
    cpt = cpt + (cpt % 2)
    e_pad = cpt * NT * CHUNK

    row = edge_index[0]
    col = edge_index[1]
    rowg = jnp.pad(row, (0, e_pad - e_edges))
    colg = jnp.pad(col, (0, e_pad - e_edges))
    rc = jnp.concatenate(
        [rowg.reshape(-1, CHUNK), colg.reshape(-1, CHUNK)], axis=1
    ).reshape(-1)

    def wqkv(l):
        return jnp.concatenate(
            [
                weights[f"l{l}_Wq"] * scale,
                weights[f"l{l}_Wk"],
                weights[f"l{l}_Wv"],
            ],
            axis=1,
        )

    def bqkv(l):
        return jnp.concatenate(
            [
                weights[f"l{l}_bq"] * scale,
                weights[f"l{l}_bk"],
                weights[f"l{l}_bv"],
            ]
        )

    def fold(wo, bo, w2, b2):
        k = wo.shape[0]
        aug = jnp.concatenate([wo, bo.reshape(1, -1)], axis=0)
        aug = jnp.concatenate(
            [aug, jnp.zeros((k + 1, 1), jnp.float32).at[k, 0].set(1.0)], axis=1
        )
        rhs = jnp.concatenate([w2, b2.reshape(1, -1)], axis=0)
        out = _matmul_bias(aug, rhs, jnp.zeros((rhs.shape[1],), jnp.float32))
        return out[:k], out[k]

    sc_attn = _make_sc_attention(n_pad, hdim, e_pad, e_edges)

    hp = jnp.pad(h, ((0, n_pad - n_nodes), (0, 0)))
    q, k, v = _proj3(hp, wqkv(0), bqkv(0), hdim)
    outv, outd = sc_attn(q, k, v, rc)
    for l in range(1, nlayers):
        aw, ab = fold(weights[f"l{l-1}_Wo"], weights[f"l{l-1}_bo"], wqkv(l), bqkv(l))
        q, k, v = _norm3(outv, outd, aw, ab, hdim)
        outv, outd = sc_attn(q, k, v, rc)
    lw = nlayers - 1
    wp = jnp.pad(weights["Wp"], ((0, 0), (0, 127)))
    bp = jnp.pad(weights["bp"], (0, 127))
    fw, fb = fold(weights[f"l{lw}_Wo"], weights[f"l{lw}_bo"], wp, bp)
    res = _normf(outv, outd, fw, fb, hdim)
    return res[:n_nodes, :1]
